# baseline (speedup 1.0000x reference)
import functools

import jax
import jax.numpy as jnp
from jax import lax
from jax.experimental import pallas as pl
from jax.experimental.pallas import tpu as pltpu
from jax.experimental.pallas import tpu_sc as plsc

NUM_GRAPHS = 128

NC = 2
NS = 16
NW = NC * NS
CHUNK = 128
FEAT = 16
KQ = 8


def _agg_loop(src_v, dst_v, table_sh, acc_sh, rows0, rows1,
              semg, sems0, sems1, nchunk):
    def wait_scatters(rows, sem):
        for t in range(KQ):
            pltpu.make_async_copy(rows.at[t], acc_sh.at[dst_v.at[0]],
                                  sem).wait()

    def body(i, carry):
        blk0 = (2 * i) * KQ
        blk1 = (2 * i + 1) * KQ
        g0 = [pltpu.async_copy(table_sh.at[src_v.at[blk0 + t]],
                               rows0.at[t], semg) for t in range(KQ)]

        @pl.when(i > 0)
        def _():
            wait_scatters(rows1, sems1)

        for d in g0:
            d.wait()
        for t in range(KQ):
            pltpu.async_copy(rows0.at[t], acc_sh.at[dst_v.at[blk0 + t]],
                             sems0, add=True)
        g1 = [pltpu.async_copy(table_sh.at[src_v.at[blk1 + t]],
                               rows1.at[t], semg) for t in range(KQ)]
        wait_scatters(rows0, sems0)
        for d in g1:
            d.wait()
        for t in range(KQ):
            pltpu.async_copy(rows1.at[t], acc_sh.at[dst_v.at[blk1 + t]],
                             sems1, add=True)
        return carry

    lax.fori_loop(0, nchunk // (2 * KQ), body, 0)
    wait_scatters(rows1, sems1)


def _sc_fused_layer1(n, n_pad, f_in, nchunk):
    mesh = plsc.VectorSubcoreMesh(core_axis_name="c", subcore_axis_name="s")
    rpt = n_pad // NS
    ndeg = (NW * nchunk) // NS

    @functools.partial(
        pl.kernel,
        out_type=[
            jax.ShapeDtypeStruct((NC, n_pad, FEAT), jnp.float32),
            jax.ShapeDtypeStruct((n_pad,), jnp.float32),
            jax.ShapeDtypeStruct((n_pad, FEAT), jnp.float32),
        ],
        mesh=mesh,
        scratch_types=[
            pltpu.VMEM((nchunk, CHUNK), jnp.int32),
            pltpu.VMEM((nchunk, CHUNK), jnp.int32),
            pltpu.VMEM((16, CHUNK), jnp.int32),
            pltpu.VMEM((CHUNK,), jnp.float32),
            pltpu.VMEM((rpt,), jnp.int32),
            pltpu.VMEM((rpt * FEAT,), jnp.int32),
            pltpu.VMEM((rpt * FEAT,), jnp.float32),
            pltpu.VMEM((rpt, FEAT), jnp.float32),
            pltpu.VMEM((rpt, FEAT), jnp.float32),
            pltpu.VMEM((rpt,), jnp.float32),
            pltpu.VMEM((rpt,), jnp.float32),
            pltpu.VMEM((KQ, CHUNK, FEAT), jnp.float32),
            pltpu.VMEM((KQ, CHUNK, FEAT), jnp.float32),
            pltpu.VMEM_SHARED((f_in, FEAT), jnp.float32),
            pltpu.VMEM_SHARED((n_pad,), jnp.float32),
            pltpu.VMEM_SHARED((n_pad,), jnp.float32),
            pltpu.VMEM_SHARED((n_pad, FEAT), jnp.float32),
            pltpu.VMEM_SHARED((n_pad, FEAT), jnp.float32),
            pltpu.SemaphoreType.DMA,
            pltpu.SemaphoreType.DMA,
            pltpu.SemaphoreType.DMA,
        ],
        compiler_params=pltpu.CompilerParams(use_tc_tiling_on_sc=False,
                                             needs_layout_passes=False),
    )
    def k(src_hbm, dst_hbm, x_hbm, w1_hbm, zeros_n_hbm, zeros_nf_hbm,
          m1_hbm, dinv_hbm, u1_hbm,
          src_v, dst_v, degdst_v, ones_v, xflat_v, repidx_v, drep_v,
          g1_v, stage_v, degline_v, dinv_v, rows0, rows1,
          w1_sh, deg_sh, dinv_sh, table_sh, acc_sh, semg, sems0, sems1):
        c = lax.axis_index("c")
        s = lax.axis_index("s")
        w = c * NS + s

        for i in range(CHUNK // 16):
            ones_v[pl.ds(i * 16, 16)] = jnp.ones((16,), jnp.float32)

        pltpu.sync_copy(zeros_n_hbm.at[pl.ds(s * rpt, rpt)], degline_v)
        pltpu.sync_copy(degline_v, deg_sh.at[pl.ds(s * rpt, rpt)])
        pltpu.sync_copy(zeros_nf_hbm.at[pl.ds(s * rpt, rpt)], stage_v)
        pltpu.sync_copy(stage_v, acc_sh.at[pl.ds(s * rpt, rpt)])

        @pl.when(s == 0)
        def _():
            pltpu.sync_copy(w1_hbm, g1_v.at[pl.ds(0, f_in)])
            pltpu.sync_copy(g1_v.at[pl.ds(0, f_in)], w1_sh)

        plsc.subcore_barrier()

        def degbody(j, carry):
            pltpu.sync_copy(dst_hbm.at[pl.ds(s * ndeg + j * 16, 16)],
                            degdst_v)
            for t in range(16):
                pltpu.sync_copy(ones_v, deg_sh.at[degdst_v.at[t]], add=True)
            return carry

        lax.fori_loop(0, ndeg // 16, degbody, 0)
        plsc.subcore_barrier()

        pltpu.sync_copy(deg_sh.at[pl.ds(s * rpt, rpt)], degline_v)

        def nbody(i, carry):
            x = degline_v[pl.ds(i * 16, 16)] + 1.0
            kbits = plsc.bitcast(x, jnp.int32)
            y = plsc.bitcast(
                jnp.int32(0x5F3759DF) - lax.shift_right_logical(
                    kbits, jnp.int32(1)), jnp.float32)
            hx = 0.5 * x
            y = y * (1.5 - hx * y * y)
            y = y * (1.5 - hx * y * y)
            y = y * (1.5 - hx * y * y)
            gid = s * rpt + i * 16 + lax.iota(jnp.int32, 16)
            dinv_v[pl.ds(i * 16, 16)] = jnp.where(gid < n, y, 0.0)
            return carry

        lax.fori_loop(0, rpt // 16, nbody, 0)
        pltpu.sync_copy(dinv_v, dinv_sh.at[pl.ds(s * rpt, rpt)])

        @pl.when(c == 0)
        def _():
            pltpu.sync_copy(dinv_v, dinv_hbm.at[pl.ds(s * rpt, rpt)])

        pltpu.sync_copy(x_hbm.at[pl.ds(s * rpt, rpt)], xflat_v)

        def repbody(i, carry):
            repidx_v[pl.ds(i * 16, 16)] = s * rpt + lax.shift_right_logical(
                i * 16 + lax.iota(jnp.int32, 16), jnp.int32(4))
            return carry

        lax.fori_loop(0, rpt * FEAT // 16, repbody, 0)
        for j in range(rpt // CHUNK):
            pltpu.async_copy(
                w1_sh.at[xflat_v.at[pl.ds(j * CHUNK, CHUNK)]],
                g1_v.at[pl.ds(j * CHUNK, CHUNK)], semg).wait()

        def dgbody(j, carry):
            pltpu.async_copy(
                dinv_sh.at[repidx_v.at[pl.ds(j * CHUNK, CHUNK)]],
                drep_v.at[pl.ds(j * CHUNK, CHUNK)], semg).wait()
            return carry

        lax.fori_loop(0, rpt * FEAT // CHUNK, dgbody, 0)

        def scbody(r, carry):
            g1_v[r, :] = g1_v[r, :] * drep_v[pl.ds(r * FEAT, FEAT)]
            return carry

        lax.fori_loop(0, rpt, scbody, 0)
        pltpu.sync_copy(g1_v, table_sh.at[pl.ds(s * rpt, rpt)])

        @pl.when(c == 0)
        def _():
            pltpu.sync_copy(g1_v, u1_hbm.at[pl.ds(s * rpt, rpt)])

        pltpu.sync_copy(src_hbm.at[pl.ds(w * nchunk, nchunk)], src_v)
        pltpu.sync_copy(dst_hbm.at[pl.ds(w * nchunk, nchunk)], dst_v)
        plsc.subcore_barrier()

        _agg_loop(src_v, dst_v, table_sh, acc_sh, rows0, rows1,
                  semg, sems0, sems1, nchunk)
        plsc.subcore_barrier()
        pltpu.sync_copy(acc_sh.at[pl.ds(s * rpt, rpt)], stage_v)
        pltpu.sync_copy(stage_v, m1_hbm.at[c, pl.ds(s * rpt, rpt)])

    return k


def _sc_edge_aggregate(n_pad, nchunk):
    mesh = plsc.VectorSubcoreMesh(core_axis_name="c", subcore_axis_name="s")
    rpt = n_pad // NS

    @functools.partial(
        pl.kernel,
        out_type=jax.ShapeDtypeStruct((NC, n_pad, FEAT), jnp.float32),
        mesh=mesh,
        scratch_types=[
            pltpu.VMEM((nchunk, CHUNK), jnp.int32),
            pltpu.VMEM((nchunk, CHUNK), jnp.int32),
            pltpu.VMEM((KQ, CHUNK, FEAT), jnp.float32),
            pltpu.VMEM((KQ, CHUNK, FEAT), jnp.float32),
            pltpu.VMEM((rpt, FEAT), jnp.float32),
            pltpu.VMEM_SHARED((n_pad, FEAT), jnp.float32),
            pltpu.VMEM_SHARED((n_pad, FEAT), jnp.float32),
            pltpu.SemaphoreType.DMA,
            pltpu.SemaphoreType.DMA,
            pltpu.SemaphoreType.DMA,
        ],
        compiler_params=pltpu.CompilerParams(use_tc_tiling_on_sc=False),
    )
    def agg_kernel(src_hbm, dst_hbm, table_hbm, zeros_hbm, out_hbm,
                   src_v, dst_v, rows0, rows1, stage_v, acc_sh, table_sh,
                   semg, sems0, sems1):
        c = lax.axis_index("c")
        s = lax.axis_index("s")
        w = c * NS + s

        pltpu.sync_copy(zeros_hbm.at[pl.ds(s * rpt, rpt)], stage_v)
        pltpu.sync_copy(stage_v, acc_sh.at[pl.ds(s * rpt, rpt)])
        pltpu.sync_copy(table_hbm.at[pl.ds(s * rpt, rpt)], stage_v)
        pltpu.sync_copy(stage_v, table_sh.at[pl.ds(s * rpt, rpt)])
        pltpu.sync_copy(src_hbm.at[pl.ds(w * nchunk, nchunk)], src_v)
        pltpu.sync_copy(dst_hbm.at[pl.ds(w * nchunk, nchunk)], dst_v)
        plsc.subcore_barrier()

        _agg_loop(src_v, dst_v, table_sh, acc_sh, rows0, rows1,
                  semg, sems0, sems1, nchunk)
        plsc.subcore_barrier()
        pltpu.sync_copy(acc_sh.at[pl.ds(s * rpt, rpt)], stage_v)
        pltpu.sync_copy(stage_v, out_hbm.at[c, pl.ds(s * rpt, rpt)])

    return agg_kernel


def _tc_mid(n_pad):
    def body(m0_ref, m1_ref, dv_ref, u1_ref, b1_ref, out_ref):
        dv = dv_ref[...]
        h1 = jnp.maximum(
            dv * (m0_ref[...] + m1_ref[...] + u1_ref[...]) + b1_ref[...], 0.0)
        out_ref[...] = dv * h1

    return pl.pallas_call(
        body, out_shape=jax.ShapeDtypeStruct((n_pad, FEAT), jnp.float32))


def _tc_final(n_pad, h2f):
    def body(m0_ref, m1_ref, dv_ref, t2_ref, w2_ref, b2_ref, batch_ref,
             wl_ref, bl_ref, out_ref):
        dv = dv_ref[...]
        z = dv * (m0_ref[...] + m1_ref[...] + t2_ref[...])
        h2 = jnp.dot(z, w2_ref[...],
                     preferred_element_type=jnp.float32) + b2_ref[...]
        kio = lax.broadcasted_iota(jnp.int32, (n_pad, NUM_GRAPHS), 1)
        bm = batch_ref[...] == kio
        acc = jnp.zeros((1, NUM_GRAPHS), jnp.float32) + bl_ref[...]
        for j in range(h2f):
            col = jnp.where(bm, h2[:, j:j + 1], -jnp.inf)
            mj = jnp.max(col, axis=0)
            acc = acc + wl_ref[:, j:j + 1] * mj[None, :]
        out_ref[...] = acc

    return pl.pallas_call(
        body, out_shape=jax.ShapeDtypeStruct((1, NUM_GRAPHS), jnp.float32))


def kernel(x, edge_index, batch, W1, b1, W2, b2, Wl, bl):
    n = x.shape[0]
    f_in = W1.shape[0]
    h2f = W2.shape[1]
    e = edge_index.shape[1]

    n_pad = -(-n // (NS * CHUNK)) * (NS * CHUNK)
    blkc = 2 * KQ
    nchunk = -(-(-(-e // (NW * CHUNK))) // blkc) * blkc
    e_pad = NW * CHUNK * nchunk

    src = edge_index[0].astype(jnp.int32)
    dst = edge_index[1].astype(jnp.int32)
    padi = jnp.full((e_pad - e,), n, jnp.int32)
    src3 = jnp.concatenate([src, padi]).reshape(NW * nchunk, CHUNK)
    dst3 = jnp.concatenate([dst, padi]).reshape(NW * nchunk, CHUNK)

    zeros_n = jnp.zeros((n_pad,), jnp.float32)
    zeros_nf = jnp.zeros((n_pad, FEAT), jnp.float32)
    xp = jnp.concatenate(
        [x.astype(jnp.int32), jnp.zeros((n_pad - n,), jnp.int32)])

    m1p, dinv, u1 = _sc_fused_layer1(n, n_pad, f_in, nchunk)(
        src3, dst3, xp, W1, zeros_n, zeros_nf)
    dv = dinv.reshape(n_pad, 1)

    b1r = b1.reshape(1, FEAT)
    table2 = _tc_mid(n_pad)(m1p[0], m1p[1], dv, u1, b1r)

    m2p = _sc_edge_aggregate(n_pad, nchunk)(src3, dst3, table2, zeros_nf)

    w2p = jnp.pad(W2, ((0, 0), (0, FEAT - h2f)))
    b2r = jnp.pad(b2, (0, FEAT - h2f)).reshape(1, FEAT)
    batchp = jnp.concatenate(
        [batch.astype(jnp.int32), jnp.full((n_pad - n,), NUM_GRAPHS, jnp.int32)]
    ).reshape(n_pad, 1)
    wlr = jnp.pad(Wl, ((0, 0), (0, FEAT - h2f)))
    blr = bl.reshape(1, 1)
    out = _tc_final(n_pad, h2f)(m2p[0], m2p[1], dv, table2, w2p, b2r,
                                batchp, wlr, blr)
    return out.reshape(NUM_GRAPHS)
